# baseline (speedup 1.0000x reference)
import math

import jax
import jax.numpy as jnp
from jax.experimental import pallas as pl
from jax.experimental.pallas import tpu as pltpu
from jax.experimental.pallas import tpu_sc as plsc

_D_MODEL = 1024
_D_FF = 4096
_E = 64
_CF = 1.0
_ALPHA = 0.01
_ZLOSS = 0.001

_FC = 512
_W = 128


def _router_body(tok_ref, rw_ref, dest_ref, comb_ref, scale_ref, aux_ref):
    t = tok_ref.shape[0]
    e = rw_ref.shape[0]
    cap = max(int(math.ceil(t / float(e) * _CF)), 1)
    dummy = e * cap

    tok = tok_ref[...]
    rw = rw_ref[...]
    logits = jax.lax.dot_general(
        tok, rw, (((1,), (1,)), ((), ())),
        preferred_element_type=jnp.float32)
    m = jnp.max(logits, axis=-1, keepdims=True)
    unnorm = jnp.exp(logits - m)
    denom = jnp.sum(unnorm, axis=-1, keepdims=True)
    probs = unnorm / denom
    top_p = jnp.max(probs, axis=-1)
    iota_e = jax.lax.broadcasted_iota(jnp.int32, (t, e), 1)
    top_i = jnp.min(jnp.where(probs == top_p[:, None], iota_e, e), axis=-1)
    oh = (top_i[:, None] == iota_e)
    oh_f = oh.astype(jnp.float32)
    oh_b = oh.astype(jnp.bfloat16)
    ir = jax.lax.broadcasted_iota(jnp.int32, (t, t), 0)
    ic = jax.lax.broadcasted_iota(jnp.int32, (t, t), 1)
    lt = (ir >= ic).astype(jnp.bfloat16)
    cum = jax.lax.dot_general(
        lt, oh_b, (((1,), (0,)), ((), ())),
        preferred_element_type=jnp.float32)
    pos = jnp.sum(cum * oh_f, axis=-1) - 1.0
    pos_i = pos.astype(jnp.int32)
    fits = pos_i < cap
    pos_clip = jnp.where(fits, pos_i, 0)
    comb = top_i * cap + pos_clip
    dest = jnp.where(fits, comb, dummy)
    scale = jnp.where(fits, top_p, 0.0)

    counts = jnp.sum(oh_f, axis=0)
    fi = counts / t
    pi = jnp.mean(probs, axis=0)
    aux = _ALPHA * e * jnp.sum(fi * pi)
    lse = m[:, 0] + jnp.log(denom[:, 0])
    z = jnp.mean(lse * lse)

    dest_ref[...] = dest[:, None]
    comb_ref[...] = comb[:, None]
    scale_ref[...] = scale[:, None]
    aux_ref[...] = jnp.broadcast_to(aux + _ZLOSS * z, (1, 1))


def _route(tokens, router_w):
    t = tokens.shape[0]
    return pl.pallas_call(
        _router_body,
        out_shape=(
            jax.ShapeDtypeStruct((t, 1), jnp.int32),
            jax.ShapeDtypeStruct((t, 1), jnp.int32),
            jax.ShapeDtypeStruct((t, 1), jnp.float32),
            jax.ShapeDtypeStruct((1, 1), jnp.float32),
        ),
    )(tokens, router_w)


def _dispatch(tokens, dest_row, n_rows):
    t, d = tokens.shape
    mesh = plsc.VectorSubcoreMesh(core_axis_name="c", subcore_axis_name="s")

    @pl.kernel(out_type=jax.ShapeDtypeStruct((n_rows, d), tokens.dtype),
               mesh=mesh)
    def k(x_hbm, i_hbm, o_hbm):
        def body(x_vmem, i_vmem):
            pltpu.sync_copy(x_vmem, o_hbm.at[i_vmem.at[0]])

        pltpu.emit_pipeline(
            body,
            grid=(t // _W,),
            in_specs=[
                pl.BlockSpec((_W, d), lambda i: (i, 0)),
                pl.BlockSpec((1, _W), lambda i: (0, i)),
            ],
            out_specs=[],
            core_axis_name=("c", "s"),
            dimension_semantics=(pltpu.PARALLEL,),
        )(x_hbm, i_hbm)

    return k(tokens, dest_row)


def _combine(expert_out, comb_row):
    t = comb_row.shape[1]
    d = expert_out.shape[1]
    mesh = plsc.VectorSubcoreMesh(core_axis_name="c", subcore_axis_name="s")

    @pl.kernel(out_type=jax.ShapeDtypeStruct((t, d), expert_out.dtype),
               mesh=mesh)
    def k(x_hbm, i_hbm, o_hbm):
        def body(i_vmem, o_vmem):
            pltpu.sync_copy(x_hbm.at[i_vmem.at[0]], o_vmem)

        pltpu.emit_pipeline(
            body,
            grid=(t // _W,),
            in_specs=[pl.BlockSpec((1, _W), lambda i: (0, i))],
            out_specs=[pl.BlockSpec((_W, d), lambda i: (i, 0))],
            core_axis_name=("c", "s"),
            dimension_semantics=(pltpu.PARALLEL,),
        )(i_hbm, o_hbm)

    return k(expert_out, comb_row)


def _ffn_body(x_ref, w1_ref, b1_ref, w2_ref, b2_ref, out_ref):
    f = pl.program_id(1)

    @pl.when(f == 0)
    def _():
        out_ref[...] = jnp.broadcast_to(b2_ref[0], out_ref.shape)

    x = x_ref[...].astype(jnp.bfloat16)
    w1 = w1_ref[0].astype(jnp.bfloat16)
    w2 = w2_ref[0].astype(jnp.bfloat16)
    h = jax.lax.dot_general(
        x, w1, (((1,), (0,)), ((), ())),
        preferred_element_type=jnp.float32)
    h = jnp.maximum(h + b1_ref[0], 0.0).astype(jnp.bfloat16)
    out_ref[...] += jax.lax.dot_general(
        h, w2, (((1,), (0,)), ((), ())),
        preferred_element_type=jnp.float32)


def _ffn(expert_in, w1, b1, w2, b2, cap):
    e = w1.shape[0]
    d = w1.shape[1]
    grid = (e, _D_FF // _FC)
    return pl.pallas_call(
        _ffn_body,
        grid=grid,
        in_specs=[
            pl.BlockSpec((cap, d), lambda i, j: (i, 0)),
            pl.BlockSpec((1, d, _FC), lambda i, j: (i, 0, j)),
            pl.BlockSpec((1, 1, _FC), lambda i, j: (i, 0, j)),
            pl.BlockSpec((1, _FC, d), lambda i, j: (i, j, 0)),
            pl.BlockSpec((1, 1, d), lambda i, j: (i, 0, 0)),
        ],
        out_specs=pl.BlockSpec((cap, d), lambda i, j: (i, 0)),
        out_shape=jax.ShapeDtypeStruct((e * cap, d), jnp.float32),
        compiler_params=pltpu.CompilerParams(
            dimension_semantics=("arbitrary", "arbitrary")),
    )(expert_in, w1, b1.reshape(e, 1, _D_FF), w2, b2.reshape(e, 1, d))


def _scale_body(g_ref, s_ref, o_ref):
    o_ref[...] = g_ref[...] * s_ref[...]


def _scale_mul(gathered, scale):
    return pl.pallas_call(
        _scale_body,
        out_shape=jax.ShapeDtypeStruct(gathered.shape, gathered.dtype),
    )(gathered, scale)


def kernel(x, router_w, w1, b1, w2, b2):
    t = x.shape[0] * x.shape[1]
    d = x.shape[2]
    e = router_w.shape[0]
    cap = max(int(math.ceil(t / float(e) * _CF)), 1)
    tokens = x.reshape(t, d)

    dest, comb, scale, aux = _route(tokens, router_w)

    nsub = d // _W
    sub = jnp.arange(nsub, dtype=jnp.int32)
    dest8 = (dest * nsub + sub).reshape(1, t * nsub)
    comb8 = (comb * nsub + sub).reshape(1, t * nsub)

    n_rows = e * cap + cap
    expert_in = _dispatch(tokens.reshape(t * nsub, _W), dest8,
                          n_rows * nsub).reshape(n_rows, d)
    expert_out = _ffn(expert_in, w1, b1, w2, b2, cap)
    gathered = _combine(expert_out.reshape(e * cap * nsub, _W),
                        comb8).reshape(t, d)
    y = _scale_mul(gathered, scale).reshape(x.shape)
    return y, aux[0, 0]


# speedup vs baseline: 1.1808x; 1.1808x over previous
import math

import jax
import jax.numpy as jnp
from jax.experimental import pallas as pl
from jax.experimental.pallas import tpu as pltpu
from jax.experimental.pallas import tpu_sc as plsc

_D_MODEL = 1024
_D_FF = 4096
_E = 64
_CF = 1.0
_ALPHA = 0.01
_ZLOSS = 0.001

_FC = 1024
_W = 128


def _router_body(tok_ref, rw_ref, dest_ref, comb_ref, scale_ref, aux_ref):
    t = tok_ref.shape[0]
    e = rw_ref.shape[0]
    cap = max(int(math.ceil(t / float(e) * _CF)), 1)
    dummy = e * cap

    tok = tok_ref[...]
    rw = rw_ref[...]
    logits = jax.lax.dot_general(
        tok, rw, (((1,), (1,)), ((), ())),
        preferred_element_type=jnp.float32)
    m = jnp.max(logits, axis=-1, keepdims=True)
    unnorm = jnp.exp(logits - m)
    denom = jnp.sum(unnorm, axis=-1, keepdims=True)
    probs = unnorm / denom
    top_p = jnp.max(probs, axis=-1)
    iota_e = jax.lax.broadcasted_iota(jnp.int32, (t, e), 1)
    top_i = jnp.min(jnp.where(probs == top_p[:, None], iota_e, e), axis=-1)
    oh = (top_i[:, None] == iota_e)
    oh_f = oh.astype(jnp.float32)
    oh_b = oh.astype(jnp.bfloat16)
    ir = jax.lax.broadcasted_iota(jnp.int32, (t, t), 0)
    ic = jax.lax.broadcasted_iota(jnp.int32, (t, t), 1)
    lt = (ir >= ic).astype(jnp.bfloat16)
    cum = jax.lax.dot_general(
        lt, oh_b, (((1,), (0,)), ((), ())),
        preferred_element_type=jnp.float32)
    pos = jnp.sum(cum * oh_f, axis=-1) - 1.0
    pos_i = pos.astype(jnp.int32)
    fits = pos_i < cap
    pos_clip = jnp.where(fits, pos_i, 0)
    comb = top_i * cap + pos_clip
    dest = jnp.where(fits, comb, dummy)
    scale = jnp.where(fits, top_p, 0.0)

    counts = jnp.sum(oh_f, axis=0)
    fi = counts / t
    pi = jnp.mean(probs, axis=0)
    aux = _ALPHA * e * jnp.sum(fi * pi)
    lse = m[:, 0] + jnp.log(denom[:, 0])
    z = jnp.mean(lse * lse)

    dest_ref[...] = dest[:, None]
    comb_ref[...] = comb[:, None]
    scale_ref[...] = scale[:, None]
    aux_ref[...] = jnp.broadcast_to(aux + _ZLOSS * z, (1, 1))


def _route(tokens, router_w):
    t = tokens.shape[0]
    return pl.pallas_call(
        _router_body,
        out_shape=(
            jax.ShapeDtypeStruct((t, 1), jnp.int32),
            jax.ShapeDtypeStruct((t, 1), jnp.int32),
            jax.ShapeDtypeStruct((t, 1), jnp.float32),
            jax.ShapeDtypeStruct((1, 1), jnp.float32),
        ),
    )(tokens, router_w)


def _dispatch(tokens, dest_row, n_rows):
    t, d = tokens.shape
    mesh = plsc.VectorSubcoreMesh(core_axis_name="c", subcore_axis_name="s")

    @pl.kernel(out_type=jax.ShapeDtypeStruct((n_rows, d), tokens.dtype),
               mesh=mesh)
    def k(x_hbm, i_hbm, o_hbm):
        def body(x_vmem, i_vmem):
            pltpu.sync_copy(x_vmem, o_hbm.at[i_vmem.at[0]])

        pltpu.emit_pipeline(
            body,
            grid=(t // _W,),
            in_specs=[
                pl.BlockSpec((_W, d), lambda i: (i, 0)),
                pl.BlockSpec((1, _W), lambda i: (0, i)),
            ],
            out_specs=[],
            core_axis_name=("c", "s"),
            dimension_semantics=(pltpu.PARALLEL,),
        )(x_hbm, i_hbm)

    return k(tokens, dest_row)


def _combine(expert_out, comb_row):
    t = comb_row.shape[1]
    d = expert_out.shape[1]
    mesh = plsc.VectorSubcoreMesh(core_axis_name="c", subcore_axis_name="s")

    @pl.kernel(out_type=jax.ShapeDtypeStruct((t, d), expert_out.dtype),
               mesh=mesh)
    def k(x_hbm, i_hbm, o_hbm):
        def body(i_vmem, o_vmem):
            pltpu.sync_copy(x_hbm.at[i_vmem.at[0]], o_vmem)

        pltpu.emit_pipeline(
            body,
            grid=(t // _W,),
            in_specs=[pl.BlockSpec((1, _W), lambda i: (0, i))],
            out_specs=[pl.BlockSpec((_W, d), lambda i: (i, 0))],
            core_axis_name=("c", "s"),
            dimension_semantics=(pltpu.PARALLEL,),
        )(i_hbm, o_hbm)

    return k(expert_out, comb_row)


def _ffn_body(x_ref, w1_ref, b1_ref, w2_ref, b2_ref, out_ref):
    f = pl.program_id(1)

    @pl.when(f == 0)
    def _():
        out_ref[...] = jnp.broadcast_to(b2_ref[0], out_ref.shape)

    x = x_ref[...].astype(jnp.bfloat16)
    w1 = w1_ref[0].astype(jnp.bfloat16)
    w2 = w2_ref[0].astype(jnp.bfloat16)
    h = jax.lax.dot_general(
        x, w1, (((1,), (0,)), ((), ())),
        preferred_element_type=jnp.float32)
    h = jnp.maximum(h + b1_ref[0], 0.0).astype(jnp.bfloat16)
    out_ref[...] += jax.lax.dot_general(
        h, w2, (((1,), (0,)), ((), ())),
        preferred_element_type=jnp.float32)


def _ffn(expert_in, w1, b1, w2, b2, cap):
    e = w1.shape[0]
    d = w1.shape[1]
    grid = (e, _D_FF // _FC)
    return pl.pallas_call(
        _ffn_body,
        grid=grid,
        in_specs=[
            pl.BlockSpec((cap, d), lambda i, j: (i, 0)),
            pl.BlockSpec((1, d, _FC), lambda i, j: (i, 0, j)),
            pl.BlockSpec((1, 1, _FC), lambda i, j: (i, 0, j)),
            pl.BlockSpec((1, _FC, d), lambda i, j: (i, j, 0)),
            pl.BlockSpec((1, 1, d), lambda i, j: (i, 0, 0)),
        ],
        out_specs=pl.BlockSpec((cap, d), lambda i, j: (i, 0)),
        out_shape=jax.ShapeDtypeStruct((e * cap, d), jnp.float32),
        compiler_params=pltpu.CompilerParams(
            dimension_semantics=("arbitrary", "arbitrary")),
    )(expert_in, w1, b1.reshape(e, 1, _D_FF), w2, b2.reshape(e, 1, d))


def _scale_body(g_ref, s_ref, o_ref):
    o_ref[...] = g_ref[...] * s_ref[...]


def _scale_mul(gathered, scale):
    return pl.pallas_call(
        _scale_body,
        out_shape=jax.ShapeDtypeStruct(gathered.shape, gathered.dtype),
    )(gathered, scale)


def kernel(x, router_w, w1, b1, w2, b2):
    t = x.shape[0] * x.shape[1]
    d = x.shape[2]
    e = router_w.shape[0]
    cap = max(int(math.ceil(t / float(e) * _CF)), 1)
    tokens = x.reshape(t, d)

    dest, comb, scale, aux = _route(tokens, router_w)

    nsub = d // _W
    sub = jnp.arange(nsub, dtype=jnp.int32)
    dest8 = (dest * nsub + sub).reshape(1, t * nsub)
    comb8 = (comb * nsub + sub).reshape(1, t * nsub)

    n_rows = e * cap + cap
    expert_in = _dispatch(tokens.reshape(t * nsub, _W), dest8,
                          n_rows * nsub).reshape(n_rows, d)
    expert_out = _ffn(expert_in, w1, b1, w2, b2, cap)
    gathered = _combine(expert_out.reshape(e * cap * nsub, _W),
                        comb8).reshape(t, d)
    y = _scale_mul(gathered, scale).reshape(x.shape)
    return y, aux[0, 0]
